# baseline (speedup 1.0000x reference)
import jax
import jax.numpy as jnp
import numpy as np
from jax.experimental import pallas as pl
from jax.experimental.pallas import tpu as pltpu

_B, _V = 128, 100000
_CW = 10000
_NC = _V // _CW
_FULL = _CW // 8

_K1 = np.uint32(42)
_K2 = np.uint32(0 ^ 42 ^ 0x1BD11BDA)
_ROT = ((13, 15, 26, 6), (17, 29, 16, 24))
_TINY = np.float32(np.finfo(np.float32).tiny)
_IMAX = np.int32(2**31 - 1)


def _threefry_bits(x1):
    ks = (np.uint32(0), _K1, _K2)
    x0 = x1
    x1 = x1 ^ ((x1 << 13) | (x1 >> 19))
    for r in _ROT[0][1:]:
        x0 = x0 + x1
        x1 = (x1 << r) | (x1 >> (32 - r))
        x1 = x0 ^ x1
    x0 = x0 + ks[1]
    x1 = x1 + np.uint32(ks[2] + 1)
    for it in range(1, 5):
        for r in _ROT[it % 2]:
            x0 = x0 + x1
            x1 = (x1 << r) | (x1 >> (32 - r))
            x1 = x0 ^ x1
        if (it + 1) % 3:
            x0 = x0 + ks[(it + 1) % 3]
        x1 = x1 + np.uint32(ks[(it + 2) % 3] + it + 1)
    return x0 ^ x1


def _gumbel_plus(logits, x1):
    bits = _threefry_bits(x1)
    fb = (bits >> 9) | jnp.uint32(0x3F800000)
    u = jax.lax.bitcast_convert_type(fb, jnp.float32) - jnp.float32(1.0)
    u = jnp.maximum(u, _TINY)
    return logits - jnp.log(-jnp.log(u))


def _scan_chunk(lt_ref, cnt, n_strips):
    vmax = jnp.full((8, _B), -jnp.inf, jnp.float32)
    vidx = jnp.zeros((8, _B), jnp.int32)
    for t in range(n_strips):
        val = _gumbel_plus(lt_ref[pl.ds(t * 8, 8), :], cnt)
        vidx = jnp.where(val > vmax, t, vidx)
        vmax = jnp.maximum(vmax, val)
        cnt = cnt + np.uint32(8)
    return vmax, vidx


def _sample_kernel(lt_ref, out_ref, smax_ref, sidx_ref):
    j = pl.program_id(0)

    @pl.when(j == 0)
    def _init():
        smax_ref[...] = jnp.full((8, _B), -jnp.inf, jnp.float32)
        sidx_ref[...] = jnp.zeros((8, _B), jnp.int32)

    cnt0 = (jax.lax.broadcasted_iota(jnp.int32, (8, _B), 1) * _V
            + jax.lax.broadcasted_iota(jnp.int32, (8, _B), 0)).astype(jnp.uint32)
    cnt0 = cnt0 + (j * _CW + 42).astype(jnp.uint32)

    def _merge(vmax, vidx):
        gidx = vidx + j * _FULL
        upd = vmax > smax_ref[...]
        sidx_ref[...] = jnp.where(upd, gidx, sidx_ref[...])
        smax_ref[...] = jnp.maximum(smax_ref[...], vmax)

    _merge(*_scan_chunk(lt_ref, cnt0, _FULL))

    @pl.when(j == _NC - 1)
    def _finalize():
        vm = smax_ref[...]
        col = sidx_ref[...] * 8 + jax.lax.broadcasted_iota(jnp.int32, (8, _B), 0)
        rmax = jnp.max(vm, axis=0, keepdims=True)
        cand = jnp.where(vm == rmax, col, _IMAX)
        out_ref[...] = jnp.min(cand, axis=0, keepdims=True)


def kernel(logits):
    out = pl.pallas_call(
        _sample_kernel,
        grid=(_NC,),
        in_specs=[pl.BlockSpec((_CW, _B), lambda j: (j, 0))],
        out_specs=pl.BlockSpec((1, _B), lambda j: (0, 0)),
        out_shape=jax.ShapeDtypeStruct((1, _B), jnp.int32),
        scratch_shapes=[
            pltpu.VMEM((8, _B), jnp.float32),
            pltpu.VMEM((8, _B), jnp.int32),
        ],
        compiler_params=pltpu.CompilerParams(
            dimension_semantics=("arbitrary",),
        ),
    )(logits.T)
    return out.reshape(_B)
